# baseline (speedup 1.0000x reference)
import math

import jax
import jax.numpy as jnp
from jax import lax
from jax.experimental import pallas as pl
from jax.experimental.pallas import tpu as pltpu
from jax.experimental.pallas import tpu_sc as plsc

_NC = 2
_NS = 16
_NW = _NC * _NS

_BATCH = 16384
_SEQ = 50
_EMB = 32
_VOCAB = 1000000
_SCALE = math.sqrt(float(_EMB))
_LANE = 128

_mesh = plsc.VectorSubcoreMesh(
    core_axis_name="c", subcore_axis_name="s", num_cores=_NC, num_subcores=_NS
)
_params = pltpu.CompilerParams(use_tc_tiling_on_sc=True, needs_layout_passes=False)


_NFULL = _VOCAB // _LANE
_VTAIL = _VOCAB - _NFULL * _LANE
_CNT_LO = _NFULL // _NW
_EXTRA = _NFULL - _CNT_LO * _NW


def _tr_body(tabt_hbm, tail_hbm, dense_hbm, vin, vout):
    wid = lax.axis_index("s") * _NC + lax.axis_index("c")
    iota = lax.iota(jnp.int32, 16)
    start = jnp.where(wid < _EXTRA, (_CNT_LO + 1) * wid, _CNT_LO * wid + _EXTRA)
    cnt = jnp.where(wid < _EXTRA, _CNT_LO + 1, _CNT_LO)

    def chunk(i, carry):
        v0 = pl.multiple_of((start + i) * _LANE, _LANE)
        pltpu.sync_copy(tabt_hbm.at[:, pl.ds(v0, _LANE)], vin)

        def row(r, carry2):
            for vv in range(4):
                col = jnp.broadcast_to(r * 4 + vv, (16,)) + iota * 0
                for h in range(2):
                    vout[r, pl.ds(vv * 32 + h * 16, 16)] = plsc.load_gather(
                        vin, [h * 16 + iota, col]
                    )
            return carry2

        lax.fori_loop(0, 32, row, 0)
        pltpu.sync_copy(vout, dense_hbm.at[pl.ds((start + i) * 32, 32)])
        return carry

    lax.fori_loop(0, cnt, chunk, 0)

    @pl.when(wid == _NW - 1)
    def _tail():
        pltpu.sync_copy(tail_hbm, dense_hbm.at[pl.ds(_NFULL * 32, 16)])


_tr_call = pl.kernel(
    _tr_body,
    out_type=jax.ShapeDtypeStruct((_VOCAB * _EMB // _LANE, _LANE), jnp.float32),
    mesh=_mesh,
    scratch_types=[
        pltpu.VMEM((_EMB, _LANE), jnp.float32),
        pltpu.VMEM((_EMB, _LANE), jnp.float32),
    ],
    compiler_params=_params,
)


_NBAND = (_SEQ + 7) // 8
_NBLK = _BATCH // _LANE
_UNITS_PER_W = _NBAND * _NBLK // _NW


def _g_body(tokt_hbm, dense_hbm, out_hbm, tok_v, gidx_v, rows_v, tile_v, sem):
    wid = lax.axis_index("s") * _NC + lax.axis_index("c")
    iota = lax.iota(jnp.int32, 16)

    def unit(k, carry):
        u = wid + _NW * k
        band = u >> 7
        s0 = pl.multiple_of(band * 8, 8)
        b0 = pl.multiple_of((u & 127) * _LANE, _LANE)
        ns = jnp.minimum(8, _SEQ - band * 8)
        pltpu.sync_copy(tokt_hbm.at[pl.ds(s0, 8), pl.ds(b0, _LANE)], tok_v)

        for j in range(8):
            for g in range(8):
                gidx_v[j, pl.ds(g * 16, 16)] = tok_v[j, pl.ds(g * 16, 16)] >> 2

        def srow(s, carry2):
            pltpu.async_copy(
                dense_hbm.at[gidx_v.at[s]], rows_v, sem
            ).wait()
            for g in range(8):
                tokv = tok_v[s, pl.ds(g * 16, 16)]
                colb = (tokv & 3) << 5
                rowv = g * 16 + iota

                def ecol(e, colv, g=g):
                    tile_v[e, pl.ds(g * 16, 16)] = (
                        plsc.load_gather(rows_v, [rowv, colv]) * _SCALE
                    )
                    return colv + 1

                lax.fori_loop(0, _EMB, ecol, colb)
            pltpu.sync_copy(tile_v, out_hbm.at[s0 + s, :, pl.ds(b0, _LANE)])
            return carry2

        lax.fori_loop(0, ns, srow, 0)
        return carry

    lax.fori_loop(0, _UNITS_PER_W, unit, 0)


_g_call = pl.kernel(
    _g_body,
    out_type=jax.ShapeDtypeStruct((_SEQ, _EMB, _BATCH), jnp.float32),
    mesh=_mesh,
    scratch_types=[
        pltpu.VMEM((8, _LANE), jnp.int32),
        pltpu.VMEM((8, _LANE), jnp.int32),
        pltpu.VMEM((_LANE, _LANE), jnp.float32),
        pltpu.VMEM((_EMB, _LANE), jnp.float32),
        pltpu.SemaphoreType.DMA,
    ],
    compiler_params=_params,
)


@jax.jit
def kernel(tokens, embedding):
    tok_t = jnp.pad(tokens, ((0, 0), (0, 56 - _SEQ))).T
    tab_t = embedding.T
    tail = embedding[_NFULL * _LANE:, :].reshape(16, _LANE)
    dense = _tr_call(tab_t, tail)
    out_t = _g_call(tok_t, dense)
    return out_t.transpose(2, 0, 1)


# speedup vs baseline: 1.2225x; 1.2225x over previous
import math

import jax
import jax.numpy as jnp
from jax import lax
from jax.experimental import pallas as pl
from jax.experimental.pallas import tpu as pltpu
from jax.experimental.pallas import tpu_sc as plsc

_NC = 2
_NS = 16
_NW = _NC * _NS

_BATCH = 16384
_SEQ = 50
_EMB = 32
_VOCAB = 1000000
_SCALE = math.sqrt(float(_EMB))
_LANE = 128

_mesh = plsc.VectorSubcoreMesh(
    core_axis_name="c", subcore_axis_name="s", num_cores=_NC, num_subcores=_NS
)
_params = pltpu.CompilerParams(use_tc_tiling_on_sc=True, needs_layout_passes=False)


_NFULL = _VOCAB // _LANE
_CH_PER_W = _NFULL // _NW
_EXTRA_W = _NFULL - _CH_PER_W * _NW


def _tr_body(tabt_hbm, tail_hbm, dense_hbm, vin0, vin1, vout0, vout1,
             sg0, sg1, so0, so1):
    wid = lax.axis_index("s") * _NC + lax.axis_index("c")
    start = wid * _CH_PER_W

    def src(ci):
        v0 = pl.multiple_of((start + ci) * _LANE, _LANE)
        return tabt_hbm.at[:, pl.ds(v0, _LANE)]

    def transpose(vb, ob):
        for r in range(32):
            for vv in range(4):
                col = jnp.full((16,), r * 4 + vv, jnp.int32)
                for h in range(2):
                    ob[r, pl.ds(vv * 32 + h * 16, 16)] = plsc.load_gather(
                        vb, [h * 16 + lax.iota(jnp.int32, 16), col]
                    )

    pltpu.async_copy(src(0), vin0, sg0)
    pltpu.async_copy(src(1), vin1, sg1)

    def pair(p, carry):
        for b, vb, ob, sg, so in ((0, vin0, vout0, sg0, so0),
                                  (1, vin1, vout1, sg1, so1)):
            ci = p * 2 + b
            pltpu.make_async_copy(src(ci), vb, sg).wait()

            @pl.when(ci >= 2)
            def _drain(ob=ob, so=so):
                pltpu.make_async_copy(ob, dense_hbm.at[pl.ds(0, 32)], so).wait()

            transpose(vb, ob)
            pltpu.async_copy(ob, dense_hbm.at[pl.ds((start + ci) * 32, 32)], so)

            @pl.when(ci + 2 < _CH_PER_W)
            def _prefetch(ci=ci, vb=vb, sg=sg):
                pltpu.async_copy(src(ci + 2), vb, sg)
        return carry

    lax.fori_loop(0, _CH_PER_W // 2, pair, 0)
    pltpu.make_async_copy(vout0, dense_hbm.at[pl.ds(0, 32)], so0).wait()
    pltpu.make_async_copy(vout1, dense_hbm.at[pl.ds(0, 32)], so1).wait()

    @pl.when(wid < _EXTRA_W)
    def _extra():
        ci = _CH_PER_W * _NW + wid - start
        pltpu.sync_copy(src(ci), vin0)
        transpose(vin0, vout0)
        pltpu.sync_copy(vout0, dense_hbm.at[pl.ds((start + ci) * 32, 32)])

    @pl.when(wid == _NW - 1)
    def _tail():
        pltpu.sync_copy(tail_hbm, dense_hbm.at[pl.ds(_NFULL * 32, 16)])


_tr_call = pl.kernel(
    _tr_body,
    out_type=jax.ShapeDtypeStruct((_VOCAB * _EMB // _LANE, _LANE), jnp.float32),
    mesh=_mesh,
    scratch_types=[
        pltpu.VMEM((_EMB, _LANE), jnp.float32),
        pltpu.VMEM((_EMB, _LANE), jnp.float32),
        pltpu.VMEM((_EMB, _LANE), jnp.float32),
        pltpu.VMEM((_EMB, _LANE), jnp.float32),
        pltpu.SemaphoreType.DMA,
        pltpu.SemaphoreType.DMA,
        pltpu.SemaphoreType.DMA,
        pltpu.SemaphoreType.DMA,
    ],
    compiler_params=_params,
)


_NBAND = (_SEQ + 7) // 8
_NBLK = _BATCH // _LANE
_UNITS_PER_W = _NBAND * _NBLK // _NW


def _g_body(tokt_hbm, dense_hbm, out_hbm, tok_v, gidx_v,
            rows0, rows1, tile0, tile1, sg0, sg1):
    wid = lax.axis_index("s") * _NC + lax.axis_index("c")
    iota = lax.iota(jnp.int32, 16)

    def unit(k, carry):
        u = wid + _NW * k
        band = u >> 7
        s0 = pl.multiple_of(band * 8, 8)
        b0 = pl.multiple_of((u & 127) * _LANE, _LANE)
        ns = jnp.minimum(8, _SEQ - band * 8)

        pltpu.sync_copy(tokt_hbm.at[pl.ds(s0, 8), pl.ds(b0, _LANE)], tok_v)
        for j in range(8):
            for g in range(8):
                gidx_v[j, pl.ds(g * 16, 16)] = tok_v[j, pl.ds(g * 16, 16)] >> 2

        pltpu.async_copy(dense_hbm.at[gidx_v.at[0]], rows0, sg0)
        pltpu.async_copy(dense_hbm.at[gidx_v.at[1]], rows1, sg1)

        def pair(p, carry2):
            for b, rb, tb, sg in ((0, rows0, tile0, sg0), (1, rows1, tile1, sg1)):
                s = p * 2 + b
                pltpu.make_async_copy(dense_hbm.at[gidx_v.at[s]], rb, sg).wait()
                for g in range(8):
                    tokv = tok_v[s, pl.ds(g * 16, 16)]
                    colv = (tokv & 3) << 5
                    rowv = g * 16 + iota
                    for e in range(_EMB):
                        tb[e, pl.ds(g * 16, 16)] = (
                            plsc.load_gather(rb, [rowv, colv]) * _SCALE
                        )
                        colv = colv + 1

                @pl.when(s + 2 < ns)
                def _prefetch(s=s, rb=rb, sg=sg):
                    pltpu.async_copy(dense_hbm.at[gidx_v.at[s + 2]], rb, sg)

                pltpu.sync_copy(tb, out_hbm.at[s0 + s, :, pl.ds(b0, _LANE)])
            return carry2

        lax.fori_loop(0, ns >> 1, pair, 0)
        return carry

    lax.fori_loop(0, _UNITS_PER_W, unit, 0)


_g_call = pl.kernel(
    _g_body,
    out_type=jax.ShapeDtypeStruct((_SEQ, _EMB, _BATCH), jnp.float32),
    mesh=_mesh,
    scratch_types=[
        pltpu.VMEM((8, _LANE), jnp.int32),
        pltpu.VMEM((8, _LANE), jnp.int32),
        pltpu.VMEM((_LANE, _LANE), jnp.float32),
        pltpu.VMEM((_LANE, _LANE), jnp.float32),
        pltpu.VMEM((_EMB, _LANE), jnp.float32),
        pltpu.VMEM((_EMB, _LANE), jnp.float32),
        pltpu.SemaphoreType.DMA,
        pltpu.SemaphoreType.DMA,
    ],
    compiler_params=_params,
)


@jax.jit
def kernel(tokens, embedding):
    tok_t = jnp.pad(tokens, ((0, 0), (0, 56 - _SEQ))).T
    tab_t = embedding.T
    tail = embedding[_NFULL * _LANE:, :].reshape(16, _LANE)
    dense = _tr_call(tab_t, tail)
    out_t = _g_call(tok_t, dense)
    return out_t.transpose(2, 0, 1)


# speedup vs baseline: 1.9003x; 1.5544x over previous
import math

import jax
import jax.numpy as jnp
from jax import lax
from jax.experimental import pallas as pl
from jax.experimental.pallas import tpu as pltpu
from jax.experimental.pallas import tpu_sc as plsc

_NC = 2
_NS = 16
_NW = _NC * _NS

_BATCH = 16384
_SEQ = 50
_EMB = 32
_TOTAL = _BATCH * _SEQ
_SCALE = math.sqrt(float(_EMB))

_LANE = 128
_TROW = 8
_CH = _TROW * _LANE
_ROWS_PER_W = _TOTAL // _NW // _LANE
_NCHUNK = _ROWS_PER_W // _TROW
_OUT_ROWS_CH = _CH * _EMB // _LANE


def _emb_body(tok_hbm, tab_hbm, out_hbm, idx_v, rows_v, out_v, sem):
    wid = lax.axis_index("s") * _NC + lax.axis_index("c")
    tok_base = wid * _ROWS_PER_W
    out_base = wid * (_ROWS_PER_W * _EMB)

    def chunk(c, carry):
        trow = tok_base + c * _TROW
        pltpu.sync_copy(tok_hbm.at[pl.ds(trow, _TROW)], idx_v)
        cps = [
            pltpu.async_copy(
                tab_hbm.at[idx_v.at[j]],
                rows_v.at[pl.ds(j * _LANE, _LANE)],
                sem,
            )
            for j in range(_TROW)
        ]
        for cp in cps:
            cp.wait()

        def scale(i, carry2):
            for u in range(4):
                for h in range(2):
                    out_v[i, pl.ds(u * 32 + h * 16, 16)] = (
                        rows_v[i * 4 + u, pl.ds(h * 16, 16)] * _SCALE
                    )
            return carry2

        lax.fori_loop(0, _OUT_ROWS_CH, scale, 0)
        pltpu.sync_copy(
            out_v, out_hbm.at[pl.ds(out_base + c * _OUT_ROWS_CH, _OUT_ROWS_CH)]
        )
        return carry

    lax.fori_loop(0, _NCHUNK, chunk, 0)


_mesh = plsc.VectorSubcoreMesh(
    core_axis_name="c", subcore_axis_name="s", num_cores=_NC, num_subcores=_NS
)

_emb_call = pl.kernel(
    _emb_body,
    out_type=jax.ShapeDtypeStruct((_TOTAL * _EMB // _LANE, _LANE), jnp.float32),
    mesh=_mesh,
    scratch_types=[
        pltpu.VMEM((_TROW, _LANE), jnp.int32),
        pltpu.VMEM((_CH, _EMB), jnp.float32),
        pltpu.VMEM((_OUT_ROWS_CH, _LANE), jnp.float32),
        pltpu.SemaphoreType.DMA,
    ],
    compiler_params=pltpu.CompilerParams(
        use_tc_tiling_on_sc=False, needs_layout_passes=False
    ),
)


@jax.jit
def kernel(tokens, embedding):
    tok = tokens.reshape(_TOTAL // _LANE, _LANE)
    out = _emb_call(tok, embedding)
    return out.reshape(_BATCH, _SEQ, _EMB)


# speedup vs baseline: 1.9501x; 1.0262x over previous
import math

import jax
import jax.numpy as jnp
from jax import lax
from jax.experimental import pallas as pl
from jax.experimental.pallas import tpu as pltpu
from jax.experimental.pallas import tpu_sc as plsc

_NC = 2
_NS = 16
_NW = _NC * _NS

_BATCH = 16384
_SEQ = 50
_EMB = 32
_TOTAL = _BATCH * _SEQ
_SCALE = math.sqrt(float(_EMB))

_LANE = 128
_TROW = 4
_CH = _TROW * _LANE
_ROWS_PER_W = _TOTAL // _NW // _LANE
_NCHUNK = _ROWS_PER_W // _TROW
_OUT_ROWS_CH = _CH * _EMB // _LANE


def _emb_body(tok_hbm, tab_hbm, out_hbm,
              idx0, idx1, rows0, rows1, out0, out1, sg0, sg1):
    wid = lax.axis_index("s") * _NC + lax.axis_index("c")
    tok_base = wid * _ROWS_PER_W
    out_base = wid * (_ROWS_PER_W * _EMB)

    def stage_and_fire(c, idx_b, rows_b, sg):
        pltpu.sync_copy(tok_hbm.at[pl.ds(tok_base + c * _TROW, _TROW)], idx_b)
        for j in range(_TROW):
            pltpu.async_copy(
                tab_hbm.at[idx_b.at[j]],
                rows_b.at[pl.ds(j * _LANE, _LANE)],
                sg,
            )

    stage_and_fire(0, idx0, rows0, sg0)
    stage_and_fire(1, idx1, rows1, sg1)

    def pair(p, carry):
        for b, idx_b, rows_b, out_b, sg in (
            (0, idx0, rows0, out0, sg0),
            (1, idx1, rows1, out1, sg1),
        ):
            c = p * 2 + b
            for j in range(_TROW):
                pltpu.make_async_copy(
                    tab_hbm.at[idx_b.at[j]],
                    rows_b.at[pl.ds(j * _LANE, _LANE)],
                    sg,
                ).wait()

            def scale(i, carry2):
                for u in range(4):
                    for h in range(2):
                        out_b[i, pl.ds(u * 32 + h * 16, 16)] = (
                            rows_b[i * 4 + u, pl.ds(h * 16, 16)] * _SCALE
                        )
                return carry2

            lax.fori_loop(0, _OUT_ROWS_CH, scale, 0)

            @pl.when(c + 2 < _NCHUNK)
            def _prefetch(c=c, idx_b=idx_b, rows_b=rows_b, sg=sg):
                stage_and_fire(c + 2, idx_b, rows_b, sg)

            pltpu.sync_copy(
                out_b,
                out_hbm.at[pl.ds(out_base + c * _OUT_ROWS_CH, _OUT_ROWS_CH)],
            )
        return carry

    lax.fori_loop(0, _NCHUNK // 2, pair, 0)


_mesh = plsc.VectorSubcoreMesh(
    core_axis_name="c", subcore_axis_name="s", num_cores=_NC, num_subcores=_NS
)

_emb_call = pl.kernel(
    _emb_body,
    out_type=jax.ShapeDtypeStruct((_TOTAL * _EMB // _LANE, _LANE), jnp.float32),
    mesh=_mesh,
    scratch_types=[
        pltpu.VMEM((_TROW, _LANE), jnp.int32),
        pltpu.VMEM((_TROW, _LANE), jnp.int32),
        pltpu.VMEM((_CH, _EMB), jnp.float32),
        pltpu.VMEM((_CH, _EMB), jnp.float32),
        pltpu.VMEM((_OUT_ROWS_CH, _LANE), jnp.float32),
        pltpu.VMEM((_OUT_ROWS_CH, _LANE), jnp.float32),
        pltpu.SemaphoreType.DMA,
        pltpu.SemaphoreType.DMA,
    ],
    compiler_params=pltpu.CompilerParams(
        use_tc_tiling_on_sc=False, needs_layout_passes=False
    ),
)


@jax.jit
def kernel(tokens, embedding):
    tok = tokens.reshape(_TOTAL // _LANE, _LANE)
    out = _emb_call(tok, embedding)
    return out.reshape(_BATCH, _SEQ, _EMB)
